# baseline (speedup 1.0000x reference)
import jax
import jax.numpy as jnp
from jax import lax
from jax.experimental import pallas as pl

PATCH = 16
FEAT = 24
NP_ = FEAT * FEAT
CUT = NP_ // 2
IMG = FEAT * PATCH


def _fix_kernel(x_ref, img_ref, out_ref):
    a_row = jnp.sum(x_ref[0, :, 0, :], axis=0, keepdims=True)
    a_row = lax.slice(a_row, (0, 1), (1, 577))

    aq = jnp.broadcast_to(a_row, (NP_, NP_))
    pi = lax.broadcasted_iota(jnp.int32, (NP_, NP_), 0)
    qi = lax.broadcasted_iota(jnp.int32, (NP_, NP_), 1)
    ident = (pi == qi).astype(jnp.float32)
    ap = jnp.sum(aq * ident, axis=1, keepdims=True)
    beats = (aq > ap) | ((aq == ap) & (qi < pi))
    rank = jnp.sum(beats.astype(jnp.float32), axis=1,
                   keepdims=True)
    sel = (rank < float(CUT)).astype(jnp.float32)

    p_g = lax.broadcasted_iota(jnp.int32, (FEAT, NP_), 1)
    i_g = lax.broadcasted_iota(jnp.int32, (FEAT, NP_), 0)
    G = ((p_g // FEAT) == i_g).astype(jnp.float32)
    p_h = lax.broadcasted_iota(jnp.int32, (NP_, FEAT), 0)
    j_h = lax.broadcasted_iota(jnp.int32, (NP_, FEAT), 1)
    H = ((p_h % FEAT) == j_h).astype(jnp.float32)
    m2 = jnp.dot(G, sel * H,
                 preferred_element_type=jnp.float32)

    k_t = lax.broadcasted_iota(jnp.int32, (FEAT, IMG), 0)
    c_t = lax.broadcasted_iota(jnp.int32, (FEAT, IMG), 1)
    ET = ((c_t // PATCH) == k_t).astype(jnp.float32)
    r_e = lax.broadcasted_iota(jnp.int32, (IMG, FEAT), 0)
    k_e = lax.broadcasted_iota(jnp.int32, (IMG, FEAT), 1)
    E = ((r_e // PATCH) == k_e).astype(jnp.float32)
    mcols = jnp.dot(m2, ET, preferred_element_type=jnp.float32)
    mimg = jnp.dot(E, mcols, preferred_element_type=jnp.float32)

    for c in range(img_ref.shape[1]):
        out_ref[0, c] = img_ref[0, c] * mimg


def kernel(x, input_images):
    B, C = input_images.shape[0], input_images.shape[1]
    out = pl.pallas_call(
        _fix_kernel,
        grid=(B,),
        in_specs=[
            pl.BlockSpec((1, x.shape[1], 8, x.shape[3]), lambda b: (b, 0, 0, 0)),
            pl.BlockSpec((1, C, IMG, IMG), lambda b: (b, 0, 0, 0)),
        ],
        out_specs=pl.BlockSpec((1, C, IMG, IMG), lambda b: (b, 0, 0, 0)),
        out_shape=jax.ShapeDtypeStruct((B, C, IMG, IMG), jnp.float32),
    )(x, input_images)
    return out.reshape(B, -1)


# speedup vs baseline: 2.9828x; 2.9828x over previous
import jax
import jax.numpy as jnp
from jax import lax
from jax.experimental import pallas as pl

PATCH = 16
FEAT = 24
NP_ = FEAT * FEAT
CUT = NP_ // 2
IMG = FEAT * PATCH


def _fix_kernel(x_ref, img_ref, out_ref):
    a_row = jnp.sum(x_ref[0], axis=0, keepdims=True)

    aq = jnp.broadcast_to(a_row, (NP_, NP_))
    pi = lax.broadcasted_iota(jnp.int32, (NP_, NP_), 0)
    qi = lax.broadcasted_iota(jnp.int32, (NP_, NP_), 1)
    ident = (pi == qi).astype(jnp.float32)
    ap = jnp.sum(aq * ident, axis=1, keepdims=True)
    beats = (aq > ap) | ((aq == ap) & (qi < pi))
    rank = jnp.sum(beats.astype(jnp.float32), axis=1,
                   keepdims=True)
    sel = (rank < float(CUT)).astype(jnp.float32)

    p_g = lax.broadcasted_iota(jnp.int32, (FEAT, NP_), 1)
    i_g = lax.broadcasted_iota(jnp.int32, (FEAT, NP_), 0)
    G = ((p_g // FEAT) == i_g).astype(jnp.float32)
    p_h = lax.broadcasted_iota(jnp.int32, (NP_, FEAT), 0)
    j_h = lax.broadcasted_iota(jnp.int32, (NP_, FEAT), 1)
    H = ((p_h % FEAT) == j_h).astype(jnp.float32)
    m2 = jnp.dot(G, sel * H,
                 preferred_element_type=jnp.float32)

    k_t = lax.broadcasted_iota(jnp.int32, (FEAT, IMG), 0)
    c_t = lax.broadcasted_iota(jnp.int32, (FEAT, IMG), 1)
    ET = ((c_t // PATCH) == k_t).astype(jnp.float32)
    r_e = lax.broadcasted_iota(jnp.int32, (IMG, FEAT), 0)
    k_e = lax.broadcasted_iota(jnp.int32, (IMG, FEAT), 1)
    E = ((r_e // PATCH) == k_e).astype(jnp.float32)
    mcols = jnp.dot(m2, ET, preferred_element_type=jnp.float32)
    mimg = jnp.dot(E, mcols, preferred_element_type=jnp.float32)

    for c in range(img_ref.shape[1]):
        out_ref[0, c] = img_ref[0, c] * mimg


def kernel(x, input_images):
    B, C = input_images.shape[0], input_images.shape[1]
    xr = x[:, :, 0, 1:]
    out = pl.pallas_call(
        _fix_kernel,
        grid=(B,),
        in_specs=[
            pl.BlockSpec((1, xr.shape[1], NP_), lambda b: (b, 0, 0)),
            pl.BlockSpec((1, C, IMG, IMG), lambda b: (b, 0, 0, 0)),
        ],
        out_specs=pl.BlockSpec((1, C, IMG, IMG), lambda b: (b, 0, 0, 0)),
        out_shape=jax.ShapeDtypeStruct((B, C, IMG, IMG), jnp.float32),
    )(xr, input_images)
    return out.reshape(B, -1)


# speedup vs baseline: 4.8710x; 1.6330x over previous
import jax
import jax.numpy as jnp
from jax import lax
from jax.experimental import pallas as pl

PATCH = 16
FEAT = 24
NP_ = FEAT * FEAT
CUT = NP_ // 2
IMG = FEAT * PATCH


def _fix_kernel(x_ref, img_ref, out_ref):
    a_row = jnp.sum(x_ref[0], axis=0, keepdims=True)

    aq = jnp.broadcast_to(a_row, (NP_, NP_))
    pi = lax.broadcasted_iota(jnp.int32, (NP_, NP_), 0)
    qi = lax.broadcasted_iota(jnp.int32, (NP_, NP_), 1)
    ident = (pi == qi).astype(jnp.float32)
    ap = jnp.sum(aq * ident, axis=1, keepdims=True)
    beats = (aq > ap) | ((aq == ap) & (qi < pi))
    rank = jnp.sum(beats.astype(jnp.float32), axis=1,
                   keepdims=True)
    sel = (rank < float(CUT)).astype(jnp.float32)

    p_g = lax.broadcasted_iota(jnp.int32, (FEAT, NP_), 1)
    i_g = lax.broadcasted_iota(jnp.int32, (FEAT, NP_), 0)
    G = ((p_g // FEAT) == i_g).astype(jnp.float32)
    p_h = lax.broadcasted_iota(jnp.int32, (NP_, FEAT), 0)
    j_h = lax.broadcasted_iota(jnp.int32, (NP_, FEAT), 1)
    H = ((p_h % FEAT) == j_h).astype(jnp.float32)
    m2 = jnp.dot(G, sel * H,
                 preferred_element_type=jnp.float32)

    k_t = lax.broadcasted_iota(jnp.int32, (FEAT, IMG), 0)
    c_t = lax.broadcasted_iota(jnp.int32, (FEAT, IMG), 1)
    ET = ((c_t // PATCH) == k_t).astype(jnp.float32)
    r_e = lax.broadcasted_iota(jnp.int32, (IMG, FEAT), 0)
    k_e = lax.broadcasted_iota(jnp.int32, (IMG, FEAT), 1)
    E = ((r_e // PATCH) == k_e).astype(jnp.float32)
    mcols = jnp.dot(m2, ET, preferred_element_type=jnp.float32)
    mimg = jnp.dot(E, mcols, preferred_element_type=jnp.float32)

    b = pl.program_id(0)
    flats = [jnp.reshape(img_ref[0, c] * mimg, (1, IMG * IMG))
             for c in range(img_ref.shape[1])]
    out_ref[pl.ds(b, 1), :] = jnp.concatenate(flats, axis=1)


def kernel(x, input_images):
    B, C = input_images.shape[0], input_images.shape[1]
    xr = x[:, :, 0, 1:]
    out = pl.pallas_call(
        _fix_kernel,
        grid=(B,),
        in_specs=[
            pl.BlockSpec((1, xr.shape[1], NP_), lambda b: (b, 0, 0)),
            pl.BlockSpec((1, C, IMG, IMG), lambda b: (b, 0, 0, 0)),
        ],
        out_specs=pl.BlockSpec((B, C * IMG * IMG), lambda b: (0, 0)),
        out_shape=jax.ShapeDtypeStruct((B, C * IMG * IMG), jnp.float32),
    )(xr, input_images)
    return out


# speedup vs baseline: 7.5284x; 1.5456x over previous
import jax
import jax.numpy as jnp
from jax import lax
from jax.experimental import pallas as pl
from jax.experimental.pallas import tpu as pltpu

PATCH = 16
FEAT = 24
NP_ = FEAT * FEAT
CUT = NP_ // 2
IMG = FEAT * PATCH
RB = 128
NRB = IMG // RB
COLW = RB * IMG


def _fix_kernel(x_ref, img_ref, out_ref, mh_ref):
    j = pl.program_id(0)

    @pl.when(j == 0)
    def _compute_masks():
        pi = lax.broadcasted_iota(jnp.int32, (NP_, NP_), 0)
        qi = lax.broadcasted_iota(jnp.int32, (NP_, NP_), 1)
        ident = (pi == qi).astype(jnp.float32)
        p_g = lax.broadcasted_iota(jnp.int32, (FEAT, NP_), 1)
        i_g = lax.broadcasted_iota(jnp.int32, (FEAT, NP_), 0)
        G = ((p_g // FEAT) == i_g).astype(jnp.float32)
        p_h = lax.broadcasted_iota(jnp.int32, (NP_, FEAT), 0)
        j_h = lax.broadcasted_iota(jnp.int32, (NP_, FEAT), 1)
        H = ((p_h % FEAT) == j_h).astype(jnp.float32)
        k_t = lax.broadcasted_iota(jnp.int32, (FEAT, IMG), 0)
        c_t = lax.broadcasted_iota(jnp.int32, (FEAT, IMG), 1)
        ET = ((c_t // PATCH) == k_t).astype(jnp.float32)

        for b in range(x_ref.shape[0]):
            a_row = jnp.sum(x_ref[b], axis=0, keepdims=True)
            aq = jnp.broadcast_to(a_row, (NP_, NP_))
            ap = jnp.sum(aq * ident, axis=1, keepdims=True)
            beats = (aq > ap) | ((aq == ap) & (qi < pi))
            rank = jnp.sum(beats.astype(jnp.float32), axis=1,
                           keepdims=True)
            sel = (rank < float(CUT)).astype(jnp.float32)
            m2 = jnp.dot(G, sel * H,
                         preferred_element_type=jnp.float32)
            mh_ref[b] = jnp.dot(m2, ET,
                                preferred_element_type=jnp.float32)

    @pl.when(j > 0)
    def _stream_block():
        jj = j - 1
        rb = jj // 3
        m8 = mh_ref[:, pl.ds(8 * rb, 8), :]
        mrows = jnp.repeat(m8, PATCH, axis=1)
        prod = img_ref[:, 0] * mrows
        out_ref[...] = jnp.reshape(prod, (prod.shape[0], COLW))


def kernel(x, input_images):
    B, C = input_images.shape[0], input_images.shape[1]
    xr = x[:, :, 0, 1:]

    def img_idx(jg):
        jj = jnp.maximum(jg - 1, 0)
        return (0, jj % C, jj // C, 0)

    def out_idx(jg):
        jj = jnp.maximum(jg - 1, 0)
        return (0, (jj % C) * NRB + jj // C)

    out = pl.pallas_call(
        _fix_kernel,
        grid=(1 + C * NRB,),
        in_specs=[
            pl.BlockSpec((B, xr.shape[1], NP_), lambda jg: (0, 0, 0)),
            pl.BlockSpec((B, 1, RB, IMG), img_idx),
        ],
        out_specs=pl.BlockSpec((B, COLW), out_idx),
        out_shape=jax.ShapeDtypeStruct((B, C * IMG * IMG), jnp.float32),
        scratch_shapes=[pltpu.VMEM((B, FEAT, IMG), jnp.float32)],
    )(xr, input_images)
    return out
